# baseline (speedup 1.0000x reference)
import functools

import jax
import jax.numpy as jnp
from jax.experimental import pallas as pl
from jax.experimental.pallas import tpu as pltpu


def _body(m_ref, a_ref, b_ref, o_ref):
    h = pl.program_id(1)
    av = a_ref[0]
    bv = b_ref[h]
    acc = jax.lax.dot_general(
        av, bv, (((1,), (1,)), ((), ())), preferred_element_type=jnp.float32
    )
    o_ref[0] = jnp.where(m_ref[...] != 0, acc, jnp.float32(0.0))


@jax.jit
def kernel(a, b, mask):
    B, H, Sq, D = a.shape
    Sk = b.shape[2]
    TQ = 256
    nq = Sq // TQ

    a3 = a.reshape(H, Sq, D)
    b3 = b.reshape(H, Sk, D)
    m2 = mask.reshape(Sq, Sk)

    out = pl.pallas_call(
        _body,
        grid=(nq, H),
        in_specs=[
            pl.BlockSpec((TQ, Sk), lambda q, h: (q, 0)),
            pl.BlockSpec((1, TQ, D), lambda q, h: (h, q, 0)),
            pl.BlockSpec((H, Sk, D), lambda q, h: (0, 0, 0)),
        ],
        out_specs=pl.BlockSpec((1, TQ, Sk), lambda q, h: (h, q, 0)),
        out_shape=jax.ShapeDtypeStruct((H, Sq, Sk), jnp.float32),
        compiler_params=pltpu.CompilerParams(
            dimension_semantics=("parallel", "arbitrary"),
        ),
    )(m2, a3, b3)
    return out.reshape(B, H, Sq, Sk)


# speedup vs baseline: 1.5181x; 1.5181x over previous
import functools

import jax
import jax.numpy as jnp
from jax.experimental import pallas as pl
from jax.experimental.pallas import tpu as pltpu


def _body(m_ref, a_ref, b_ref, o_ref):
    h = pl.program_id(1)
    av = a_ref[0]
    bv = b_ref[h]
    acc = jax.lax.dot_general(
        av, bv, (((1,), (1,)), ((), ())), preferred_element_type=jnp.float32
    )
    o_ref[0] = jnp.where(m_ref[...] != 0, acc, jnp.float32(0.0))


@jax.jit
def kernel(a, b, mask):
    B, H, Sq, D = a.shape
    Sk = b.shape[2]
    TQ = 1024
    nq = Sq // TQ

    a3 = a.reshape(H, Sq, D)
    b3 = b.reshape(H, Sk, D)
    m2 = mask.reshape(Sq, Sk)

    out = pl.pallas_call(
        _body,
        grid=(nq, H),
        in_specs=[
            pl.BlockSpec((TQ, Sk), lambda q, h: (q, 0)),
            pl.BlockSpec((1, TQ, D), lambda q, h: (h, q, 0)),
            pl.BlockSpec((H, Sk, D), lambda q, h: (0, 0, 0)),
        ],
        out_specs=pl.BlockSpec((1, TQ, Sk), lambda q, h: (h, q, 0)),
        out_shape=jax.ShapeDtypeStruct((H, Sq, Sk), jnp.float32),
        compiler_params=pltpu.CompilerParams(
            dimension_semantics=("parallel", "arbitrary"),
        ),
    )(m2, a3, b3)
    return out.reshape(B, H, Sq, Sk)


# speedup vs baseline: 1.5207x; 1.0017x over previous
import functools

import jax
import jax.numpy as jnp
from jax.experimental import pallas as pl
from jax.experimental.pallas import tpu as pltpu


def _body(m_ref, a_ref, b_ref, o_ref):
    q = pl.program_id(1)
    TQ = a_ref.shape[1]
    av = a_ref[0]
    bv = b_ref[0]
    acc = jax.lax.dot_general(
        av, bv, (((1,), (1,)), ((), ())), preferred_element_type=jnp.float32
    )
    m = m_ref[pl.ds(q * TQ, TQ), :]
    o_ref[0] = jnp.where(m != 0, acc, jnp.float32(0.0))


@jax.jit
def kernel(a, b, mask):
    B, H, Sq, D = a.shape
    Sk = b.shape[2]
    TQ = 1024
    nq = Sq // TQ

    a3 = a.reshape(H, Sq, D)
    b3 = b.reshape(H, Sk, D)
    m2 = mask.reshape(Sq, Sk)

    out = pl.pallas_call(
        _body,
        grid=(H, nq),
        in_specs=[
            pl.BlockSpec((Sq, Sk), lambda h, q: (0, 0)),
            pl.BlockSpec((1, TQ, D), lambda h, q: (h, q, 0)),
            pl.BlockSpec((1, Sk, D), lambda h, q: (h, 0, 0)),
        ],
        out_specs=pl.BlockSpec((1, TQ, Sk), lambda h, q: (h, q, 0)),
        out_shape=jax.ShapeDtypeStruct((H, Sq, Sk), jnp.float32),
        compiler_params=pltpu.CompilerParams(
            dimension_semantics=("parallel", "parallel"),
        ),
    )(m2, a3, b3)
    return out.reshape(B, H, Sq, Sk)


# speedup vs baseline: 1.5456x; 1.0164x over previous
import functools

import jax
import jax.numpy as jnp
from jax.experimental import pallas as pl
from jax.experimental.pallas import tpu as pltpu


def _body(m_ref, a_ref, b_ref, o_ref):
    q = pl.program_id(1)
    TQ = a_ref.shape[1]
    av = a_ref[0]
    bv = b_ref[0]
    acc = jax.lax.dot_general(
        av, bv, (((1,), (1,)), ((), ())), preferred_element_type=jnp.float32
    )
    m = m_ref[pl.ds(q * TQ, TQ), :]
    o_ref[0] = jnp.where(m != 0, acc, jnp.float32(0.0))


@jax.jit
def kernel(a, b, mask):
    B, H, Sq, D = a.shape
    Sk = b.shape[2]
    TQ = 2048
    nq = Sq // TQ

    a3 = a.reshape(H, Sq, D)
    b3 = b.reshape(H, Sk, D)
    m2 = mask.reshape(Sq, Sk)

    out = pl.pallas_call(
        _body,
        grid=(H, nq),
        in_specs=[
            pl.BlockSpec((Sq, Sk), lambda h, q: (0, 0)),
            pl.BlockSpec((1, TQ, D), lambda h, q: (h, q, 0)),
            pl.BlockSpec((1, Sk, D), lambda h, q: (h, 0, 0)),
        ],
        out_specs=pl.BlockSpec((1, TQ, Sk), lambda h, q: (h, q, 0)),
        out_shape=jax.ShapeDtypeStruct((H, Sq, Sk), jnp.float32),
        compiler_params=pltpu.CompilerParams(
            dimension_semantics=("parallel", "parallel"),
        ),
    )(m2, a3, b3)
    return out.reshape(B, H, Sq, Sk)
